# Initial kernel scaffold; baseline (speedup 1.0000x reference)
import functools

import jax
import jax.numpy as jnp
from jax import lax
from jax.experimental import pallas as pl
from jax.experimental.pallas import tpu as pltpu
from jax.experimental.pallas import tpu_sc as plsc

NC = 2
NS = 16
NW = NC * NS
CHUNK = 128


def _emb_body(nchunk, table_hbm, idx_hbm, out_hbm, idx_v, rows_v, sem):
    cid = lax.axis_index("c")
    sid = lax.axis_index("s")
    wid = sid * NC + cid
    pltpu.sync_copy(idx_hbm.at[wid], idx_v)

    def body(j, carry):
        pltpu.async_copy(table_hbm.at[idx_v.at[j]], rows_v, sem).wait()
        pltpu.sync_copy(rows_v, out_hbm.at[wid * nchunk + j])
        return carry

    lax.fori_loop(0, nchunk, body, 0)


def kernel(x, table):
    b, s = x.shape
    v, d = table.shape
    total = b * s
    assert total % (NW * CHUNK) == 0
    nchunk = total // (NW * CHUNK)

    idx = x.reshape(NW, nchunk, CHUNK).astype(jnp.int32)

    grid_kernel = pl.kernel(
        functools.partial(_emb_body, nchunk),
        mesh=plsc.VectorSubcoreMesh(core_axis_name="c", subcore_axis_name="s"),
        out_type=jax.ShapeDtypeStruct((NW * nchunk, CHUNK, d), jnp.float32),
        scratch_types=[
            pltpu.VMEM((nchunk, CHUNK), jnp.int32),
            pltpu.VMEM((CHUNK, d), jnp.float32),
            pltpu.SemaphoreType.DMA,
        ],
    )

    out = grid_kernel(table, idx)
    return out.reshape(b, s, d)


# speedup vs baseline: 4.0871x; 4.0871x over previous
import functools

import jax
import jax.numpy as jnp
from jax import lax
from jax.experimental import pallas as pl
from jax.experimental.pallas import tpu as pltpu
from jax.experimental.pallas import tpu_sc as plsc

NC = 2
NS = 16
NW = NC * NS
CHUNK = 128


def _emb_body(nchunk, table_hbm, idx_hbm, out_hbm, idx_v, rows_v, sem):
    cid = lax.axis_index("c")
    sid = lax.axis_index("s")
    wid = sid * NC + cid
    pltpu.sync_copy(idx_hbm.at[wid], idx_v)

    def body(j, carry):
        pltpu.async_copy(table_hbm.at[idx_v.at[j]], rows_v, sem).wait()
        pltpu.sync_copy(rows_v, out_hbm.at[wid * nchunk + j])
        return carry

    lax.fori_loop(0, nchunk, body, 0)


def kernel(x, table):
    b, s = x.shape
    v, d = table.shape
    total = b * s
    assert total % (NW * CHUNK) == 0
    nchunk = total // (NW * CHUNK)

    idx = x.reshape(NW, nchunk, CHUNK).astype(jnp.int32)

    grid_kernel = pl.kernel(
        functools.partial(_emb_body, nchunk),
        mesh=plsc.VectorSubcoreMesh(core_axis_name="c", subcore_axis_name="s"),
        out_type=jax.ShapeDtypeStruct((NW * nchunk, CHUNK, d), jnp.float32),
        scratch_types=[
            pltpu.VMEM((nchunk, CHUNK), jnp.int32),
            pltpu.VMEM((CHUNK, d), jnp.float32),
            pltpu.SemaphoreType.DMA,
        ],
        compiler_params=pltpu.CompilerParams(use_tc_tiling_on_sc=False),
    )

    out = grid_kernel(table, idx)
    return out.reshape(b, s, d)


# speedup vs baseline: 4.2603x; 1.0424x over previous
import functools

import jax
import jax.numpy as jnp
from jax import lax
from jax.experimental import pallas as pl
from jax.experimental.pallas import tpu as pltpu
from jax.experimental.pallas import tpu_sc as plsc

NC = 2
NS = 16
NW = NC * NS
CHUNK = 128


def _emb_body(nchunk, table_hbm, idx_hbm, out_hbm, idx_v, rows_v, gsem, osem):
    cid = lax.axis_index("c")
    sid = lax.axis_index("s")
    wid = sid * NC + cid
    pltpu.sync_copy(idx_hbm.at[wid], idx_v)

    pltpu.async_copy(table_hbm.at[idx_v.at[0]], rows_v.at[0], gsem)

    def body(t, carry):
        slot = lax.rem(t, 2)
        pltpu.make_async_copy(
            table_hbm.at[idx_v.at[t]], rows_v.at[slot], gsem).wait()

        @pl.when(t >= 1)
        def _():
            pltpu.make_async_copy(rows_v.at[slot], out_hbm.at[0], osem).wait()

        @pl.when(t + 1 < nchunk)
        def _():
            pltpu.async_copy(
                table_hbm.at[idx_v.at[t + 1]], rows_v.at[1 - slot], gsem)

        pltpu.async_copy(rows_v.at[slot], out_hbm.at[wid * nchunk + t], osem)
        return carry

    lax.fori_loop(0, nchunk, body, 0)
    pltpu.make_async_copy(rows_v.at[0], out_hbm.at[0], osem).wait()


def kernel(x, table):
    b, s = x.shape
    v, d = table.shape
    total = b * s
    assert total % (NW * CHUNK) == 0
    nchunk = total // (NW * CHUNK)

    idx = x.reshape(NW, nchunk, CHUNK).astype(jnp.int32)

    grid_kernel = pl.kernel(
        functools.partial(_emb_body, nchunk),
        mesh=plsc.VectorSubcoreMesh(core_axis_name="c", subcore_axis_name="s"),
        out_type=jax.ShapeDtypeStruct((NW * nchunk, CHUNK, d), jnp.float32),
        scratch_types=[
            pltpu.VMEM((nchunk, CHUNK), jnp.int32),
            pltpu.VMEM((2, CHUNK, d), jnp.float32),
            pltpu.SemaphoreType.DMA,
            pltpu.SemaphoreType.DMA,
        ],
        compiler_params=pltpu.CompilerParams(use_tc_tiling_on_sc=False),
    )

    out = grid_kernel(table, idx)
    return out.reshape(b, s, d)


# speedup vs baseline: 4.6859x; 1.0999x over previous
import functools

import jax
import jax.numpy as jnp
from jax import lax
from jax.experimental import pallas as pl
from jax.experimental.pallas import tpu as pltpu
from jax.experimental.pallas import tpu_sc as plsc

NC = 2
NS = 16
NW = NC * NS
CHUNK = 128
NBUF = 4


def _emb_body(nchunk, table_hbm, idx_hbm, out_hbm, idx_v, rows_v, gsem, osem):
    cid = lax.axis_index("c")
    sid = lax.axis_index("s")
    wid = sid * NC + cid
    pltpu.sync_copy(idx_hbm.at[wid], idx_v)

    for i in range(min(NBUF - 1, nchunk)):
        pltpu.async_copy(table_hbm.at[idx_v.at[i]], rows_v.at[i], gsem)

    def body(t, carry):
        slot = lax.rem(t, NBUF)
        pltpu.make_async_copy(
            table_hbm.at[idx_v.at[t]], rows_v.at[slot], gsem).wait()

        @pl.when(t >= 1)
        def _():
            pltpu.make_async_copy(rows_v.at[slot], out_hbm.at[0], osem).wait()

        @pl.when(t + NBUF - 1 < nchunk)
        def _():
            pltpu.async_copy(
                table_hbm.at[idx_v.at[t + NBUF - 1]],
                rows_v.at[lax.rem(t + NBUF - 1, NBUF)], gsem)

        pltpu.async_copy(rows_v.at[slot], out_hbm.at[wid * nchunk + t], osem)
        return carry

    lax.fori_loop(0, nchunk, body, 0)
    pltpu.make_async_copy(rows_v.at[0], out_hbm.at[0], osem).wait()


def kernel(x, table):
    b, s = x.shape
    v, d = table.shape
    total = b * s
    assert total % (NW * CHUNK) == 0
    nchunk = total // (NW * CHUNK)

    idx = x.reshape(NW, nchunk, CHUNK).astype(jnp.int32)

    grid_kernel = pl.kernel(
        functools.partial(_emb_body, nchunk),
        mesh=plsc.VectorSubcoreMesh(core_axis_name="c", subcore_axis_name="s"),
        out_type=jax.ShapeDtypeStruct((NW * nchunk, CHUNK, d), jnp.float32),
        scratch_types=[
            pltpu.VMEM((nchunk, CHUNK), jnp.int32),
            pltpu.VMEM((NBUF, CHUNK, d), jnp.float32),
            pltpu.SemaphoreType.DMA,
            pltpu.SemaphoreType.DMA,
        ],
        compiler_params=pltpu.CompilerParams(use_tc_tiling_on_sc=False),
    )

    out = grid_kernel(table, idx)
    return out.reshape(b, s, d)
